# baseline (device time: 38772 ns/iter reference)
import numpy as np
import jax
import jax.numpy as jnp
from jax import lax
from jax.experimental import pallas as pl
from jax.experimental.pallas import tpu as pltpu

N_DEV = 4
Dh = 64


def kernel(x, Wq, Wk, Wv, Wo):
    B, Sq, D = x.shape
    HD = Wq.shape[1]
    Hl = HD // Dh
    M = B * Sq

    inv = 1.0 / (10000.0 ** (np.arange(0, Dh, 2) / Dh))
    pos = np.arange(Sq)[:, None] * inv[None, :]
    cos_np = np.repeat(np.cos(pos), 2, axis=-1).astype(np.float32)
    sin_np = np.repeat(np.sin(pos), 2, axis=-1).astype(np.float32)
    R_np = np.zeros((Dh, Dh), np.float32)
    ev = np.arange(0, Dh, 2)
    R_np[ev + 1, ev] = -1.0
    R_np[ev, ev + 1] = 1.0

    def body(x_ref, wq_ref, wk_ref, wv_ref, wo_ref, cos_ref, sin_ref, rot_ref,
             out_ref, comm_ref, send_sems, recv_sems):
        my = lax.axis_index("i")
        left = lax.rem(my + N_DEV - 1, N_DEV)
        right = lax.rem(my + 1, N_DEV)

        barrier = pltpu.get_barrier_semaphore()
        for nbr in (left, right):
            pl.semaphore_signal(barrier, inc=1, device_id=(nbr,),
                                device_id_type=pl.DeviceIdType.MESH)
        pl.semaphore_wait(barrier, 2)

        x2 = x_ref[:]
        q = jnp.dot(x2, wq_ref[:], preferred_element_type=jnp.float32)
        k = jnp.dot(x2, wk_ref[:], preferred_element_type=jnp.float32)
        v = jnp.dot(x2, wv_ref[:], preferred_element_type=jnp.float32)

        cos = cos_ref[:]
        sin = sin_ref[:]
        R = rot_ref[:]

        ctx_cols = []
        for h in range(Hl):
            outs = []
            for b in range(B):
                r0 = b * Sq
                qh = q[r0:r0 + Sq, h * Dh:(h + 1) * Dh]
                kh = k[r0:r0 + Sq, h * Dh:(h + 1) * Dh]
                vh = v[r0:r0 + Sq, h * Dh:(h + 1) * Dh]
                qr = qh * cos + jnp.dot(qh, R) * sin
                kr = kh * cos + jnp.dot(kh, R) * sin
                s = lax.dot_general(
                    qr, kr, (((1,), (1,)), ((), ())),
                    preferred_element_type=jnp.float32) * 0.125
                s = s - jnp.max(s, axis=-1, keepdims=True)
                w = jnp.exp(s)
                w = w / jnp.sum(w, axis=-1, keepdims=True)
                outs.append(jnp.dot(w, vh, preferred_element_type=jnp.float32))
            ctx_cols.append(jnp.concatenate(outs, axis=0))
        ctx = jnp.concatenate(ctx_cols, axis=1)
        partial = jnp.dot(ctx, wo_ref[:],
                          preferred_element_type=jnp.float32)

        comm_ref[0] = partial
        out_ref[:] = partial
        for h in range(N_DEV - 1):
            rdma = pltpu.make_async_remote_copy(
                src_ref=comm_ref.at[h],
                dst_ref=comm_ref.at[h + 1],
                send_sem=send_sems.at[h],
                recv_sem=recv_sems.at[h],
                device_id=(right,),
                device_id_type=pl.DeviceIdType.MESH,
            )
            rdma.start()
            rdma.wait()
            out_ref[:] += comm_ref[h + 1]

    out2 = pl.pallas_call(
        body,
        out_shape=jax.ShapeDtypeStruct((M, D), jnp.float32),
        in_specs=[pl.BlockSpec(memory_space=pltpu.VMEM)] * 8,
        out_specs=pl.BlockSpec(memory_space=pltpu.VMEM),
        scratch_shapes=[
            pltpu.VMEM((N_DEV, M, D), jnp.float32),
            pltpu.SemaphoreType.DMA((N_DEV - 1,)),
            pltpu.SemaphoreType.DMA((N_DEV - 1,)),
        ],
        compiler_params=pltpu.CompilerParams(collective_id=0),
    )(
        x.reshape(M, D), Wq, Wk, Wv, Wo,
        jnp.asarray(cos_np), jnp.asarray(sin_np), jnp.asarray(R_np),
    )
    return out2.reshape(B, Sq, D)


# device time: 19383 ns/iter; 2.0003x vs baseline; 2.0003x over previous
import numpy as np
import jax
import jax.numpy as jnp
from jax import lax
from jax.experimental import pallas as pl
from jax.experimental.pallas import tpu as pltpu

N_DEV = 4
Dh = 64
bf = jnp.bfloat16


def kernel(x, Wq, Wk, Wv, Wo):
    B, Sq, D = x.shape
    HD = Wq.shape[1]
    Hl = HD // Dh
    M = B * Sq

    def body(x_ref, wq_ref, wk_ref, wv_ref, wo_ref, out_ref,
             s1_ref, s2_ref, r1_ref, r2_ref, send_sems, recv_sems):
        my = lax.axis_index("i")
        p1 = my ^ 1
        p2 = 3 - my

        barrier = pltpu.get_barrier_semaphore()
        for nbr in (p1, p2):
            pl.semaphore_signal(barrier, inc=1, device_id=(nbr,),
                                device_id_type=pl.DeviceIdType.MESH)
        pl.semaphore_wait(barrier, 2)

        srow = lax.broadcasted_iota(jnp.int32, (M, HD), 0) % Sq
        d = lax.broadcasted_iota(jnp.int32, (M, HD), 1) % Dh
        pair = (d // 2).astype(jnp.float32)
        invf = jnp.exp(pair * jnp.float32(-2.0 * np.log(10000.0) / Dh))
        ang = srow.astype(jnp.float32) * invf
        cos = jnp.cos(ang)
        sin = jnp.sin(ang)
        rr = lax.broadcasted_iota(jnp.int32, (HD, HD), 0)
        cc = lax.broadcasted_iota(jnp.int32, (HD, HD), 1)
        same = (rr // Dh) == (cc // Dh)
        ri = rr % Dh
        ci = cc % Dh
        Rb = (jnp.where(same & (ri == ci + 1) & (ci % 2 == 0), -1.0, 0.0)
              + jnp.where(same & (ci == ri + 1) & (ri % 2 == 0), 1.0, 0.0)
              ).astype(bf)

        xb = x_ref[:].astype(bf)
        q = jnp.dot(xb, wq_ref[:].astype(bf), preferred_element_type=jnp.float32)
        k = jnp.dot(xb, wk_ref[:].astype(bf), preferred_element_type=jnp.float32)
        v = jnp.dot(xb, wv_ref[:].astype(bf), preferred_element_type=jnp.float32)
        qr = q * cos + jnp.dot(q.astype(bf), Rb,
                               preferred_element_type=jnp.float32) * sin
        kr = k * cos + jnp.dot(k.astype(bf), Rb,
                               preferred_element_type=jnp.float32) * sin
        wob = wo_ref[:].astype(bf)

        def exchange(stage, b, src_ref, dst_ref, partner):
            return pltpu.make_async_remote_copy(
                src_ref=src_ref.at[b],
                dst_ref=dst_ref.at[b],
                send_sem=send_sems.at[stage, b],
                recv_sem=recv_sems.at[stage, b],
                device_id=(partner,),
                device_id_type=pl.DeviceIdType.MESH,
            )

        rd1 = []
        for b in range(B):
            r0 = b * Sq
            cols = []
            for h in range(Hl):
                c0 = h * Dh
                qh = qr[r0:r0 + Sq, c0:c0 + Dh].astype(bf)
                kh = kr[r0:r0 + Sq, c0:c0 + Dh].astype(bf)
                vh = v[r0:r0 + Sq, c0:c0 + Dh].astype(bf)
                s = lax.dot_general(
                    qh, kh, (((1,), (1,)), ((), ())),
                    preferred_element_type=jnp.float32) * 0.125
                e = jnp.exp(s)
                denom = jnp.sum(e, axis=-1, keepdims=True)
                cols.append(jnp.dot(e.astype(bf), vh,
                                    preferred_element_type=jnp.float32) / denom)
            ctxb = jnp.concatenate(cols, axis=1)
            pb = jnp.dot(ctxb.astype(bf), wob,
                         preferred_element_type=jnp.float32)
            out_ref[r0:r0 + Sq, :] = pb
            s1_ref[b] = pb.astype(bf)
            rd = exchange(0, b, s1_ref, r1_ref, p1)
            rd.start()
            rd1.append(rd)

        rd2 = []
        for b in range(B):
            r0 = b * Sq
            rd1[b].wait_recv()
            sum2 = out_ref[r0:r0 + Sq, :] + r1_ref[b].astype(jnp.float32)
            out_ref[r0:r0 + Sq, :] = sum2
            s2_ref[b] = sum2.astype(bf)
            rd = exchange(1, b, s2_ref, r2_ref, p2)
            rd.start()
            rd2.append(rd)

        for b in range(B):
            r0 = b * Sq
            rd2[b].wait_recv()
            out_ref[r0:r0 + Sq, :] += r2_ref[b].astype(jnp.float32)

        for rd in rd1 + rd2:
            rd.wait_send()

    out2 = pl.pallas_call(
        body,
        out_shape=jax.ShapeDtypeStruct((M, D), jnp.float32),
        in_specs=[pl.BlockSpec(memory_space=pltpu.VMEM)] * 5,
        out_specs=pl.BlockSpec(memory_space=pltpu.VMEM),
        scratch_shapes=[
            pltpu.VMEM((B, Sq, D), bf),
            pltpu.VMEM((B, Sq, D), bf),
            pltpu.VMEM((B, Sq, D), bf),
            pltpu.VMEM((B, Sq, D), bf),
            pltpu.SemaphoreType.DMA((2, B)),
            pltpu.SemaphoreType.DMA((2, B)),
        ],
        compiler_params=pltpu.CompilerParams(collective_id=0),
    )(x.reshape(M, D), Wq, Wk, Wv, Wo)
    return out2.reshape(B, Sq, D)


# device time: 17941 ns/iter; 2.1611x vs baseline; 1.0804x over previous
import numpy as np
import jax
import jax.numpy as jnp
from jax import lax
from jax.experimental import pallas as pl
from jax.experimental.pallas import tpu as pltpu

N_DEV = 4
Dh = 64
bf = jnp.bfloat16


def kernel(x, Wq, Wk, Wv, Wo):
    B, Sq, D = x.shape
    HD = Wq.shape[1]
    Hl = HD // Dh

    def body(x_ref, wq_ref, wk_ref, wv_ref, wo_ref, out_ref,
             sa_ref, sb_ref, ra_ref, rb_ref, send_sems, recv_sems):
        my = lax.axis_index("i")
        p1 = my ^ 1
        p2 = 3 - my

        barrier = pltpu.get_barrier_semaphore()
        for nbr in (p1, p2):
            pl.semaphore_signal(barrier, inc=1, device_id=(nbr,),
                                device_id_type=pl.DeviceIdType.MESH)

        srow = lax.broadcasted_iota(jnp.int32, (Sq, HD), 0)
        d = lax.broadcasted_iota(jnp.int32, (Sq, HD), 1) % Dh
        pair = (d // 2).astype(jnp.float32)
        invf = jnp.exp(pair * jnp.float32(-2.0 * np.log(10000.0) / Dh))
        ang = srow.astype(jnp.float32) * invf
        cos = jnp.cos(ang)
        sin = jnp.sin(ang)
        rr = lax.broadcasted_iota(jnp.int32, (HD, HD), 0)
        cc = lax.broadcasted_iota(jnp.int32, (HD, HD), 1)
        same = (rr // Dh) == (cc // Dh)
        ri = rr % Dh
        ci = cc % Dh
        Rb = (jnp.where(same & (ri == ci + 1) & (ci % 2 == 0), -1.0, 0.0)
              + jnp.where(same & (ci == ri + 1) & (ri % 2 == 0), 1.0, 0.0)
              ).astype(bf)

        wqb = wq_ref[:].astype(bf)
        wkb = wk_ref[:].astype(bf)
        wvb = wv_ref[:].astype(bf)
        wob = wo_ref[:].astype(bf)

        def exchange(stage, b, src_ref, dst_ref, partner):
            return pltpu.make_async_remote_copy(
                src_ref=src_ref.at[b],
                dst_ref=dst_ref.at[b],
                send_sem=send_sems.at[stage, b],
                recv_sem=recv_sems.at[stage, b],
                device_id=(partner,),
                device_id_type=pl.DeviceIdType.MESH,
            )

        partA = (p1, p2)
        partB = (p2, p1)

        rdA = []
        for b in range(B):
            xb = x_ref[b].astype(bf)
            q = jnp.dot(xb, wqb, preferred_element_type=jnp.float32)
            k = jnp.dot(xb, wkb, preferred_element_type=jnp.float32)
            v = jnp.dot(xb, wvb, preferred_element_type=jnp.float32)
            qr = q * cos + jnp.dot(q.astype(bf), Rb,
                                   preferred_element_type=jnp.float32) * sin
            kr = k * cos + jnp.dot(k.astype(bf), Rb,
                                   preferred_element_type=jnp.float32) * sin
            cols = []
            for h in range(Hl):
                c0 = h * Dh
                qh = qr[:, c0:c0 + Dh].astype(bf)
                kh = kr[:, c0:c0 + Dh].astype(bf)
                vh = v[:, c0:c0 + Dh].astype(bf)
                s = lax.dot_general(
                    qh, kh, (((1,), (1,)), ((), ())),
                    preferred_element_type=jnp.float32) * 0.125
                e = jnp.exp(s)
                denom = jnp.sum(e, axis=-1, keepdims=True)
                cols.append(jnp.dot(e.astype(bf), vh,
                                    preferred_element_type=jnp.float32) / denom)
            ctxb = jnp.concatenate(cols, axis=1)
            pb = jnp.dot(ctxb.astype(bf), wob,
                         preferred_element_type=jnp.float32)
            out_ref[b] = pb
            sa_ref[b] = pb.astype(bf)
            if b == 0:
                pl.semaphore_wait(barrier, 2)
            rd = exchange(0, b, sa_ref, ra_ref, partA[b])
            rd.start()
            rdA.append(rd)

        rdB = []
        for b in range(B):
            rdA[b].wait_recv()
            sum2 = out_ref[b] + ra_ref[b].astype(jnp.float32)
            out_ref[b] = sum2
            sb_ref[b] = sum2.astype(bf)
            rd = exchange(1, b, sb_ref, rb_ref, partB[b])
            rd.start()
            rdB.append(rd)

        for b in range(B):
            rdB[b].wait_recv()
            out_ref[b] += rb_ref[b].astype(jnp.float32)

        for rd in rdA + rdB:
            rd.wait_send()

    return pl.pallas_call(
        body,
        out_shape=jax.ShapeDtypeStruct((B, Sq, D), jnp.float32),
        in_specs=[pl.BlockSpec(memory_space=pltpu.VMEM)] * 5,
        out_specs=pl.BlockSpec(memory_space=pltpu.VMEM),
        scratch_shapes=[
            pltpu.VMEM((B, Sq, D), bf),
            pltpu.VMEM((B, Sq, D), bf),
            pltpu.VMEM((B, Sq, D), bf),
            pltpu.VMEM((B, Sq, D), bf),
            pltpu.SemaphoreType.DMA((2, B)),
            pltpu.SemaphoreType.DMA((2, B)),
        ],
        compiler_params=pltpu.CompilerParams(collective_id=0),
    )(x, Wq, Wk, Wv, Wo)


# device time: 17014 ns/iter; 2.2788x vs baseline; 1.0545x over previous
import numpy as np
import jax
import jax.numpy as jnp
from jax import lax
from jax.experimental import pallas as pl
from jax.experimental.pallas import tpu as pltpu

N_DEV = 4
Dh = 64
bf = jnp.bfloat16


def kernel(x, Wq, Wk, Wv, Wo):
    B, Sq, D = x.shape
    HD = Wq.shape[1]
    Hl = HD // Dh
    NBLK = 2 * B
    R = Sq // 2

    def body(x_ref, wq_ref, wk_ref, wv_ref, wo_ref, out_ref,
             acc_ref, sa_ref, sb_ref, ra_ref, rb_ref,
             send_sems, recv_sems, out_sems):
        my = lax.axis_index("i")
        p1 = my ^ 1
        p2 = 3 - my

        barrier = pltpu.get_barrier_semaphore()
        for nbr in (p1, p2):
            pl.semaphore_signal(barrier, inc=1, device_id=(nbr,),
                                device_id_type=pl.DeviceIdType.MESH)

        srow = lax.broadcasted_iota(jnp.int32, (Sq, HD), 0)
        d = lax.broadcasted_iota(jnp.int32, (Sq, HD), 1) % Dh
        pair = (d // 2).astype(jnp.float32)
        invf = jnp.exp(pair * jnp.float32(-2.0 * np.log(10000.0) / Dh))
        ang = srow.astype(jnp.float32) * invf
        cos = jnp.cos(ang)
        sin = jnp.sin(ang)
        rr = lax.broadcasted_iota(jnp.int32, (HD, HD), 0)
        cc = lax.broadcasted_iota(jnp.int32, (HD, HD), 1)
        same = (rr // Dh) == (cc // Dh)
        ri = rr % Dh
        ci = cc % Dh
        Rb = (jnp.where(same & (ri == ci + 1) & (ci % 2 == 0), -1.0, 0.0)
              + jnp.where(same & (ci == ri + 1) & (ri % 2 == 0), 1.0, 0.0)
              ).astype(bf)

        wqb = wq_ref[:].astype(bf)
        wkb = wk_ref[:].astype(bf)
        wvb = wv_ref[:].astype(bf)
        wob = wo_ref[:].astype(bf)

        def exchange(stage, blk, src_ref, dst_ref, partner):
            return pltpu.make_async_remote_copy(
                src_ref=src_ref.at[blk],
                dst_ref=dst_ref.at[blk],
                send_sem=send_sems.at[stage, blk],
                recv_sem=recv_sems.at[stage, blk],
                device_id=(partner,),
                device_id_type=pl.DeviceIdType.MESH,
            )

        def partners(blk):
            return (p1, p2) if blk % 2 == 0 else (p2, p1)

        rdA = []
        for b in range(B):
            xb = x_ref[b].astype(bf)
            q = jnp.dot(xb, wqb, preferred_element_type=jnp.float32)
            k = jnp.dot(xb, wkb, preferred_element_type=jnp.float32)
            v = jnp.dot(xb, wvb, preferred_element_type=jnp.float32)
            qr = q * cos + jnp.dot(q.astype(bf), Rb,
                                   preferred_element_type=jnp.float32) * sin
            kr = k * cos + jnp.dot(k.astype(bf), Rb,
                                   preferred_element_type=jnp.float32) * sin
            cols = []
            for h in range(Hl):
                c0 = h * Dh
                qh = qr[:, c0:c0 + Dh].astype(bf)
                kh = kr[:, c0:c0 + Dh].astype(bf)
                vh = v[:, c0:c0 + Dh].astype(bf)
                s = lax.dot_general(
                    qh, kh, (((1,), (1,)), ((), ())),
                    preferred_element_type=jnp.float32) * 0.125
                e = jnp.exp(s)
                denom = jnp.sum(e, axis=-1, keepdims=True)
                cols.append(jnp.dot(e.astype(bf), vh,
                                    preferred_element_type=jnp.float32) / denom)
            ctxb = jnp.concatenate(cols, axis=1)
            pb = jnp.dot(ctxb.astype(bf), wob,
                         preferred_element_type=jnp.float32)
            for h in range(2):
                blk = 2 * b + h
                pblk = pb[h * R:(h + 1) * R, :]
                acc_ref[blk] = pblk
                sa_ref[blk] = pblk.astype(bf)
                if blk == 0:
                    pl.semaphore_wait(barrier, 2)
                rd = exchange(0, blk, sa_ref, ra_ref, partners(blk)[0])
                rd.start()
                rdA.append(rd)

        rdB = []
        for blk in range(NBLK):
            rdA[blk].wait_recv()
            sum2 = acc_ref[blk] + ra_ref[blk].astype(jnp.float32)
            acc_ref[blk] = sum2
            sb_ref[blk] = sum2.astype(bf)
            rd = exchange(1, blk, sb_ref, rb_ref, partners(blk)[1])
            rd.start()
            rdB.append(rd)

        outcps = []
        for blk in range(NBLK):
            rdB[blk].wait_recv()
            acc_ref[blk] += rb_ref[blk].astype(jnp.float32)
            b, h = divmod(blk, 2)
            cp = pltpu.make_async_copy(
                acc_ref.at[blk],
                out_ref.at[b, pl.ds(h * R, R), :],
                out_sems.at[blk],
            )
            cp.start()
            outcps.append(cp)

        for cp in outcps:
            cp.wait()
        for rd in rdA + rdB:
            rd.wait_send()

    return pl.pallas_call(
        body,
        out_shape=jax.ShapeDtypeStruct((B, Sq, D), jnp.float32),
        in_specs=[pl.BlockSpec(memory_space=pltpu.VMEM)] * 5,
        out_specs=pl.BlockSpec(memory_space=pl.ANY),
        scratch_shapes=[
            pltpu.VMEM((NBLK, R, D), jnp.float32),
            pltpu.VMEM((NBLK, R, D), bf),
            pltpu.VMEM((NBLK, R, D), bf),
            pltpu.VMEM((NBLK, R, D), bf),
            pltpu.VMEM((NBLK, R, D), bf),
            pltpu.SemaphoreType.DMA((2, NBLK)),
            pltpu.SemaphoreType.DMA((2, NBLK)),
            pltpu.SemaphoreType.DMA((NBLK,)),
        ],
        compiler_params=pltpu.CompilerParams(collective_id=0),
    )(x, Wq, Wk, Wv, Wo)
